# Initial kernel scaffold; baseline (speedup 1.0000x reference)
import jax
import jax.numpy as jnp
from jax.experimental import pallas as pl


def kernel(gate_logits):
    raise NotImplementedError("write your pallas kernel here")


# speedup vs baseline: 1.0185x; 1.0185x over previous
import functools

import jax
import jax.numpy as jnp
from jax.experimental import pallas as pl
from jax.experimental.pallas import tpu as pltpu

TOKENS = 32768
EXPERTS = 64
K = 8
ROWS_PER_BLOCK = 2048


def _topk_body(x_ref, ids_ref, vals_ref):
    x = x_ref[...].astype(jnp.float32)
    m = jnp.max(x, axis=1, keepdims=True)
    e = jnp.exp(x - m)
    s = jnp.sum(e, axis=1, keepdims=True)
    p = e / s

    lane = jax.lax.broadcasted_iota(jnp.int32, p.shape, 1)
    col = jax.lax.broadcasted_iota(jnp.int32, (p.shape[0], K), 1)
    ids = jnp.zeros((p.shape[0], K), jnp.int32)
    vals = jnp.zeros((p.shape[0], K), jnp.float32)
    work = p
    for k in range(K):
        cur = jnp.max(work, axis=1, keepdims=True)
        hit = jnp.where(work == cur, lane, EXPERTS)
        idx = jnp.min(hit, axis=1, keepdims=True)
        ids = jnp.where(col == k, idx, ids)
        vals = jnp.where(col == k, cur, vals)
        work = jnp.where(lane == idx, -1.0, work)
    ids_ref[...] = ids
    vals_ref[...] = vals.astype(jnp.bfloat16)


@jax.jit
def kernel(gate_logits):
    grid = (TOKENS // ROWS_PER_BLOCK,)
    ids, vals = pl.pallas_call(
        _topk_body,
        grid=grid,
        in_specs=[pl.BlockSpec((ROWS_PER_BLOCK, EXPERTS), lambda i: (i, 0))],
        out_specs=[
            pl.BlockSpec((ROWS_PER_BLOCK, K), lambda i: (i, 0)),
            pl.BlockSpec((ROWS_PER_BLOCK, K), lambda i: (i, 0)),
        ],
        out_shape=[
            jax.ShapeDtypeStruct((TOKENS, K), jnp.int32),
            jax.ShapeDtypeStruct((TOKENS, K), jnp.bfloat16),
        ],
    )(gate_logits)
    return (ids, vals)


# speedup vs baseline: 8.4369x; 8.2840x over previous
import jax
import jax.numpy as jnp
from jax.experimental import pallas as pl

TOKENS = 32768
EXPERTS = 64
K = 8
COLS_PER_BLOCK = 2048

_SIGN_FIX = 0x7FFF0000
_LOW_MASK = -65536


def _topk_body(xt_ref, ids_ref, vals_ref):
    xf = xt_ref[...].astype(jnp.float32)
    b = jax.lax.bitcast_convert_type(xf, jnp.int32)
    key = jnp.where(b >= 0, b, b ^ _SIGN_FIX)
    eidx = jax.lax.broadcasted_iota(jnp.int32, key.shape, 0)
    key = key + (EXPERTS - 1 - eidx)

    row = jax.lax.broadcasted_iota(jnp.int32, (K, key.shape[1]), 0)
    kstack = jnp.zeros((K, key.shape[1]), jnp.int32)
    work = key
    for k in range(K):
        kmax = jnp.max(work, axis=0, keepdims=True)
        work = jnp.where(work == kmax, jnp.int32(-(2**31)), work)
        kstack = jnp.where(row == k, jnp.broadcast_to(kmax, kstack.shape), kstack)

    ids = (EXPERTS - 1) - (kstack & (EXPERTS - 1))
    kb = kstack & _LOW_MASK
    bsel = jnp.where(kb >= 0, kb, kb ^ _SIGN_FIX)
    lsel = jax.lax.bitcast_convert_type(bsel, jnp.float32)
    m = lsel[0:1, :]
    s = jnp.sum(jnp.exp(xf - m), axis=0, keepdims=True)
    vals = jnp.exp(lsel - m) / s

    ids_ref[...] = ids
    vals_ref[...] = vals.astype(jnp.bfloat16)


@jax.jit
def kernel(gate_logits):
    xt = gate_logits.T
    grid = (TOKENS // COLS_PER_BLOCK,)
    ids_t, vals_t = pl.pallas_call(
        _topk_body,
        grid=grid,
        in_specs=[pl.BlockSpec((EXPERTS, COLS_PER_BLOCK), lambda i: (0, i))],
        out_specs=[
            pl.BlockSpec((K, COLS_PER_BLOCK), lambda i: (0, i)),
            pl.BlockSpec((K, COLS_PER_BLOCK), lambda i: (0, i)),
        ],
        out_shape=[
            jax.ShapeDtypeStruct((K, TOKENS), jnp.int32),
            jax.ShapeDtypeStruct((K, TOKENS), jnp.bfloat16),
        ],
    )(xt)
    return (ids_t.T, vals_t.T)
